# baseline (speedup 1.0000x reference)
import functools

import jax
import jax.numpy as jnp
from jax import lax
from jax.experimental import pallas as pl
from jax.experimental.pallas import tpu as pltpu
from jax.experimental.pallas import tpu_sc as plsc

VOCAB = 1000000
EMBED = 64
B = 4096
L = 200

_info = plsc.get_sparse_core_info()
NC, NS, LANES = _info.num_cores, _info.num_subcores, _info.num_lanes
NW = NC * NS
NBLK = VOCAB // 128


def _diag_transpose(src, dst, qts, qcs, r, t0, c0):
    lanes = lax.iota(jnp.int32, LANES)
    t_l = lanes + t0
    c_l = lax.rem(lanes + r, LANES) + c0
    v = plsc.load_gather(src, [t_l, c_l])
    plsc.store_scatter(dst, [c_l, t_l], v)


def _k1_body(tableT, tail128, scratch, src_v, dst_v, sem):
    wid = lax.axis_index("s") * NC + lax.axis_index("c")
    nper = NBLK // NW

    def transpose_slab(_):
        def diag(r, c):
            for qt in range(8):
                for qc in range(4):
                    lanes = lax.iota(jnp.int32, LANES)
                    c_l = lanes + 16 * qc
                    t_l = lax.rem(lanes + r, LANES) + 16 * qt
                    v = plsc.load_gather(src_v, [c_l, t_l])
                    plsc.store_scatter(dst_v, [t_l, c_l], v)
            return c

        lax.fori_loop(0, LANES, diag, 0)

    def do_block(j, carry):
        pltpu.async_copy(tableT.at[:, pl.ds(j * 128, 128)], src_v, sem).wait()
        transpose_slab(None)
        pltpu.async_copy(dst_v, scratch.at[pl.ds(j * 128, 128), :], sem).wait()
        return carry

    lax.fori_loop(wid * nper, (wid + 1) * nper, do_block, 0)

    @pl.when(wid < NBLK - NW * nper)
    def _tail_full():
        do_block(NW * nper + wid, 0)

    @pl.when(wid == NW - 1)
    def _tail_rem():
        pltpu.async_copy(tail128, src_v, sem).wait()
        transpose_slab(None)
        pltpu.async_copy(dst_v, scratch.at[pl.ds(VOCAB - 128, 128), :],
                         sem).wait()


def _k2_body(tokensT, scratch, pe, out, idx_v, rows_v, dst_v, pe_v, sem):
    wid = lax.axis_index("s") * NC + lax.axis_index("c")
    b0 = wid * 128
    pltpu.sync_copy(pe, pe_v)

    def per_l(l, _):
        pltpu.async_copy(tokensT.at[l, pl.ds(b0, 128)], idx_v, sem).wait()
        pltpu.async_copy(scratch.at[idx_v], rows_v, sem).wait()

        pe0 = pe_v[l, pl.ds(0, 16)]
        pe1 = pe_v[l, pl.ds(16, 16)]
        pe2 = pe_v[l, pl.ds(32, 16)]
        pe3 = pe_v[l, pl.ds(48, 16)]

        def add_pe(t, c):
            rows_v[t, pl.ds(0, 16)] = rows_v[t, pl.ds(0, 16)] + pe0
            rows_v[t, pl.ds(16, 16)] = rows_v[t, pl.ds(16, 16)] + pe1
            rows_v[t, pl.ds(32, 16)] = rows_v[t, pl.ds(32, 16)] + pe2
            rows_v[t, pl.ds(48, 16)] = rows_v[t, pl.ds(48, 16)] + pe3
            return c

        lax.fori_loop(0, 128, add_pe, 0)

        def diag(r, c):
            for qt in range(8):
                for qc in range(4):
                    lanes = lax.iota(jnp.int32, LANES)
                    t_l = lanes + 16 * qt
                    c_l = lax.rem(lanes + r, LANES) + 16 * qc
                    v = plsc.load_gather(rows_v, [t_l, c_l])
                    plsc.store_scatter(dst_v, [c_l, t_l], v)
            return c

        lax.fori_loop(0, LANES, diag, 0)
        pltpu.async_copy(dst_v, out.at[l, :, pl.ds(b0, 128)], sem).wait()
        return 0

    lax.fori_loop(0, L, per_l, 0)


@jax.jit
def kernel(tokens, table, pe):
    mesh = plsc.VectorSubcoreMesh(core_axis_name="c", subcore_axis_name="s")
    params = pltpu.CompilerParams(
        use_tc_tiling_on_sc=True, needs_layout_passes=False)

    k1 = functools.partial(
        pl.kernel, mesh=mesh,
        out_type=jax.ShapeDtypeStruct((VOCAB, 128), jnp.float32),
        scratch_types=[
            pltpu.VMEM((64, 128), jnp.float32),
            pltpu.VMEM((128, 128), jnp.float32),
            pltpu.SemaphoreType.DMA,
        ],
        compiler_params=params,
    )(_k1_body)
    tableT = table.T
    tail128 = lax.slice(tableT, (0, VOCAB - 128), (EMBED, VOCAB))
    scratch = k1(tableT, tail128)

    k2 = functools.partial(
        pl.kernel, mesh=mesh,
        out_type=jax.ShapeDtypeStruct((L, EMBED, B), jnp.float32),
        scratch_types=[
            pltpu.VMEM((128,), jnp.int32),
            pltpu.VMEM((128, 128), jnp.float32),
            pltpu.VMEM((EMBED, 128), jnp.float32),
            pltpu.VMEM((L, EMBED), jnp.float32),
            pltpu.SemaphoreType.DMA,
        ],
        compiler_params=params,
    )(_k2_body)
    out3 = k2(tokens.T, scratch, pe)
    return jnp.transpose(out3, (2, 0, 1))


# speedup vs baseline: 1.1618x; 1.1618x over previous
import functools

import jax
import jax.numpy as jnp
from jax import lax
from jax.experimental import pallas as pl
from jax.experimental.pallas import tpu as pltpu
from jax.experimental.pallas import tpu_sc as plsc

VOCAB = 1000000
EMBED = 64
B = 4096
L = 200

_info = plsc.get_sparse_core_info()
NC, NS, LANES = _info.num_cores, _info.num_subcores, _info.num_lanes
NW = NC * NS
NBLK = VOCAB // 128
NPER = NBLK // NW


def _k1_transpose(src_v, dst_v):
    def diag(r, carry):
        c_rot = lax.rem(lax.iota(jnp.int32, LANES) + r, LANES)
        for qt in range(8):
            t_l = c_rot + 16 * qt
            u_l = lax.shift_right_logical(t_l, 1)
            h_l = lax.shift_left(lax.bitwise_and(t_l, 1), 6)
            for qc in range(4):
                c_l = lax.iota(jnp.int32, LANES) + 16 * qc
                v = plsc.load_gather(src_v, [c_l, t_l])
                plsc.store_scatter(dst_v, [u_l, h_l + c_l], v)
        return carry

    lax.fori_loop(0, LANES, diag, 0)


def _k1_body(tableT, tail128, scratch, src0, src1, dst0, dst1,
             si0, si1, so0, so1):
    wid = lax.axis_index("s") * NC + lax.axis_index("c")
    base = wid * NPER
    srcs, dsts = (src0, src1), (dst0, dst1)
    sis, sos = (si0, si1), (so0, so1)

    def in_copy(j, p):
        return pltpu.make_async_copy(
            tableT.at[:, pl.ds(j * 128, 128)], srcs[p], sis[p])

    def out_copy(j, p):
        return pltpu.make_async_copy(
            dsts[p], scratch.at[pl.ds(j * 64, 64), :], sos[p])

    in_copy(base, 0).start()
    in_copy(base + 1, 1).start()

    def step(j, p):
        in_copy(j, p).wait()

        @pl.when(j - base >= 2)
        def _():
            out_copy(j - 2, p).wait()

        _k1_transpose(srcs[p], dsts[p])
        out_copy(j, p).start()

        @pl.when(j + 2 < base + NPER)
        def _():
            in_copy(j + 2, p).start()

    def body(i, carry):
        step(base + 2 * i, 0)
        step(base + 2 * i + 1, 1)
        return carry

    lax.fori_loop(0, NPER // 2, body, 0)
    out_copy(base + NPER - 2, 0).wait()
    out_copy(base + NPER - 1, 1).wait()

    @pl.when(wid < NBLK - NW * NPER)
    def _tail_full():
        j = NW * NPER + wid
        in_copy(j, 0).start()
        in_copy(j, 0).wait()
        _k1_transpose(src0, dst0)
        out_copy(j, 0).start()
        out_copy(j, 0).wait()

    @pl.when(wid == NW - 1)
    def _tail_rem():
        pltpu.make_async_copy(tail128, src0, si0).start()
        pltpu.make_async_copy(tail128, src0, si0).wait()
        _k1_transpose(src0, dst0)
        cp = pltpu.make_async_copy(
            dst0, scratch.at[pl.ds(VOCAB // 2 - 64, 64), :], so0)
        cp.start()
        cp.wait()


def _k2_compute(rows_v, dst_v, par_v, pe_v, lm):
    lm_splat = jnp.full((LANES,), lm, jnp.int32)

    def diag(r, carry):
        c_rot = lax.rem(lax.iota(jnp.int32, LANES) + r, LANES)
        for qc in range(4):
            c_l = c_rot + 16 * qc
            pe_d = plsc.load_gather(pe_v, [lm_splat, c_l])
            for qt in range(8):
                t_l = lax.iota(jnp.int32, LANES) + 16 * qt
                par = par_v[pl.ds(16 * qt, 16)]
                v = plsc.load_gather(rows_v, [t_l, c_l + par])
                plsc.store_scatter(dst_v, [c_l, t_l], v + pe_d)
        return carry

    lax.fori_loop(0, LANES, diag, 0)


def _k2_body(tokensT, scratch, pe, out, idx0, idx1, ix0, ix1, pr0, pr1,
             rows0, rows1, dst0, dst1, pe_v, si0, si1, sg0, sg1, so0, so1):
    wid = lax.axis_index("s") * NC + lax.axis_index("c")
    b0 = wid * 128
    idxs, ixs, prs = (idx0, idx1), (ix0, ix1), (pr0, pr1)
    rows, dsts = (rows0, rows1), (dst0, dst1)
    sis, sgs, sos = (si0, si1), (sg0, sg1), (so0, so1)

    pltpu.sync_copy(pe, pe_v)

    def idx_copy(l, p):
        return pltpu.make_async_copy(
            tokensT.at[l, pl.ds(b0, 128)], idxs[p], sis[p])

    def gather_copy(p):
        return pltpu.make_async_copy(scratch.at[ixs[p]], rows[p], sgs[p])

    def out_copy(l, p):
        return pltpu.make_async_copy(
            dsts[p], out.at[l, :, pl.ds(b0, 128)], sos[p])

    idx_copy(0, 0).start()
    idx_copy(1, 1).start()

    def arrive(l, p):
        idx_copy(l, p).wait()
        for q in range(8):
            raw = idxs[p][pl.ds(16 * q, 16)]
            ixs[p][pl.ds(16 * q, 16)] = lax.shift_right_logical(raw, 1)
            prs[p][pl.ds(16 * q, 16)] = lax.shift_left(
                lax.bitwise_and(raw, 1), 6)
        gather_copy(p).start()

        @pl.when(l + 2 < L)
        def _():
            idx_copy(l + 2, p).start()

    def compute(lm, p):
        gather_copy(p).wait()

        @pl.when(lm >= 2)
        def _():
            out_copy(lm - 2, p).wait()

        _k2_compute(rows[p], dsts[p], prs[p], pe_v, lm)
        out_copy(lm, p).start()

    def body(i, carry):
        l = 2 * i
        arrive(l, 0)

        @pl.when(l >= 1)
        def _():
            compute(l - 1, 1)

        arrive(l + 1, 1)
        compute(l, 0)
        return carry

    lax.fori_loop(0, L // 2, body, 0)
    compute(L - 1, 1)
    out_copy(L - 2, 0).wait()
    out_copy(L - 1, 1).wait()


@jax.jit
def kernel(tokens, table, pe):
    mesh = plsc.VectorSubcoreMesh(core_axis_name="c", subcore_axis_name="s")
    params = pltpu.CompilerParams(
        use_tc_tiling_on_sc=True, needs_layout_passes=False)

    k1 = functools.partial(
        pl.kernel, mesh=mesh,
        out_type=jax.ShapeDtypeStruct((VOCAB // 2, 128), jnp.float32),
        scratch_types=[
            pltpu.VMEM((EMBED, 128), jnp.float32),
            pltpu.VMEM((EMBED, 128), jnp.float32),
            pltpu.VMEM((EMBED, 128), jnp.float32),
            pltpu.VMEM((EMBED, 128), jnp.float32),
            pltpu.SemaphoreType.DMA,
            pltpu.SemaphoreType.DMA,
            pltpu.SemaphoreType.DMA,
            pltpu.SemaphoreType.DMA,
        ],
        compiler_params=params,
    )(_k1_body)
    tableT = table.T
    tail128 = lax.slice(tableT, (0, VOCAB - 128), (EMBED, VOCAB))
    scratch = k1(tableT, tail128)

    k2 = functools.partial(
        pl.kernel, mesh=mesh,
        out_type=jax.ShapeDtypeStruct((L, EMBED, B), jnp.float32),
        scratch_types=[
            pltpu.VMEM((128,), jnp.int32),
            pltpu.VMEM((128,), jnp.int32),
            pltpu.VMEM((128,), jnp.int32),
            pltpu.VMEM((128,), jnp.int32),
            pltpu.VMEM((128,), jnp.int32),
            pltpu.VMEM((128,), jnp.int32),
            pltpu.VMEM((128, 128), jnp.float32),
            pltpu.VMEM((128, 128), jnp.float32),
            pltpu.VMEM((EMBED, 128), jnp.float32),
            pltpu.VMEM((EMBED, 128), jnp.float32),
            pltpu.VMEM((L, EMBED), jnp.float32),
            pltpu.SemaphoreType.DMA,
            pltpu.SemaphoreType.DMA,
            pltpu.SemaphoreType.DMA,
            pltpu.SemaphoreType.DMA,
            pltpu.SemaphoreType.DMA,
            pltpu.SemaphoreType.DMA,
        ],
        compiler_params=params,
    )(_k2_body)
    out3 = k2(tokens.T, scratch, pe)
    return jnp.transpose(out3, (2, 0, 1))


# speedup vs baseline: 1.9074x; 1.6418x over previous
import functools

import jax
import jax.numpy as jnp
from jax import lax
from jax.experimental import pallas as pl
from jax.experimental.pallas import tpu as pltpu
from jax.experimental.pallas import tpu_sc as plsc

VOCAB = 1000000
EMBED = 64
B = 4096
L = 200

_info = plsc.get_sparse_core_info()
NC, NS, LANES = _info.num_cores, _info.num_subcores, _info.num_lanes
NW = NC * NS
NBLK = VOCAB // 128
NPER = NBLK // NW


def _k1_transpose(src_v, dst_v):
    def diag4(rb, carry):
        for dr in range(4):
            r = 4 * rb + dr
            c_rot = lax.rem(lax.iota(jnp.int32, LANES) + r, LANES)
            for qt in range(8):
                t_l = c_rot + 16 * qt
                u_l = lax.shift_right_logical(t_l, 1)
                h_l = lax.shift_left(lax.bitwise_and(t_l, 1), 6)
                for qc in range(4):
                    c_l = lax.iota(jnp.int32, LANES) + 16 * qc
                    v = plsc.load_gather(src_v, [c_l, t_l])
                    plsc.store_scatter(dst_v, [u_l, h_l + c_l], v)
        return carry

    lax.fori_loop(0, 4, diag4, 0)


def _k1_body(tableT, tail128, scratch, src0, src1, dst0, dst1,
             si0, si1, so0, so1):
    wid = lax.axis_index("s") * NC + lax.axis_index("c")
    base = wid * NPER
    srcs, dsts = (src0, src1), (dst0, dst1)
    sis, sos = (si0, si1), (so0, so1)

    def in_copy(j, p):
        return pltpu.make_async_copy(
            tableT.at[:, pl.ds(j * 128, 128)], srcs[p], sis[p])

    def out_copy(j, p):
        return pltpu.make_async_copy(
            dsts[p], scratch.at[pl.ds(j * 64, 64), :], sos[p])

    in_copy(base, 0).start()
    in_copy(base + 1, 1).start()

    def step(j, p):
        in_copy(j, p).wait()

        @pl.when(j - base >= 2)
        def _():
            out_copy(j - 2, p).wait()

        _k1_transpose(srcs[p], dsts[p])
        out_copy(j, p).start()

        @pl.when(j + 2 < base + NPER)
        def _():
            in_copy(j + 2, p).start()

    def body(i, carry):
        step(base + 2 * i, 0)
        step(base + 2 * i + 1, 1)
        return carry

    lax.fori_loop(0, NPER // 2, body, 0)
    out_copy(base + NPER - 2, 0).wait()
    out_copy(base + NPER - 1, 1).wait()

    @pl.when(wid < NBLK - NW * NPER)
    def _tail_full():
        j = NW * NPER + wid
        in_copy(j, 0).start()
        in_copy(j, 0).wait()
        _k1_transpose(src0, dst0)
        out_copy(j, 0).start()
        out_copy(j, 0).wait()

    @pl.when(wid == NW - 1)
    def _tail_rem():
        pltpu.make_async_copy(tail128, src0, si0).start()
        pltpu.make_async_copy(tail128, src0, si0).wait()
        _k1_transpose(src0, dst0)
        cp = pltpu.make_async_copy(
            dst0, scratch.at[pl.ds(VOCAB // 2 - 64, 64), :], so0)
        cp.start()
        cp.wait()


def _k2_compute(rows_v, dst_v, par_v, pe_v, lm):
    lm_splat = jnp.full((LANES,), lm, jnp.int32)

    pars = [par_v[pl.ds(16 * qt, 16)] for qt in range(8)]

    def diag4(rb, carry):
        for dr in range(4):
            r = 4 * rb + dr
            c_rot = lax.rem(lax.iota(jnp.int32, LANES) + r, LANES)
            for qc in range(4):
                c_l = c_rot + 16 * qc
                pe_d = plsc.load_gather(pe_v, [lm_splat, c_l])
                for qt in range(8):
                    t_l = lax.iota(jnp.int32, LANES) + 16 * qt
                    v = plsc.load_gather(rows_v, [t_l, c_l + pars[qt]])
                    plsc.store_scatter(dst_v, [c_l, t_l], v + pe_d)
        return carry

    lax.fori_loop(0, 4, diag4, 0)


def _k2_body(tokensT, scratch, pe, out, idx0, idx1, ix0, ix1, pr0, pr1,
             rows0, rows1, dst0, dst1, pe_v, si0, si1, sg0, sg1, so0, so1):
    wid = lax.axis_index("s") * NC + lax.axis_index("c")
    b0 = wid * 128
    idxs, ixs, prs = (idx0, idx1), (ix0, ix1), (pr0, pr1)
    rows, dsts = (rows0, rows1), (dst0, dst1)
    sis, sgs, sos = (si0, si1), (sg0, sg1), (so0, so1)

    pltpu.sync_copy(pe, pe_v)

    def idx_copy(l, p):
        return pltpu.make_async_copy(
            tokensT.at[l, pl.ds(b0, 128)], idxs[p], sis[p])

    def gather_copy(p):
        return pltpu.make_async_copy(scratch.at[ixs[p]], rows[p], sgs[p])

    def out_copy(l, p):
        return pltpu.make_async_copy(
            dsts[p], out.at[l, :, pl.ds(b0, 128)], sos[p])

    idx_copy(0, 0).start()
    idx_copy(1, 1).start()

    def arrive(l, p):
        idx_copy(l, p).wait()
        for q in range(8):
            raw = idxs[p][pl.ds(16 * q, 16)]
            ixs[p][pl.ds(16 * q, 16)] = lax.shift_right_logical(raw, 1)
            prs[p][pl.ds(16 * q, 16)] = lax.shift_left(
                lax.bitwise_and(raw, 1), 6)
        gather_copy(p).start()

        @pl.when(l + 2 < L)
        def _():
            idx_copy(l + 2, p).start()

    def compute(lm, p):
        gather_copy(p).wait()

        @pl.when(lm >= 2)
        def _():
            out_copy(lm - 2, p).wait()

        _k2_compute(rows[p], dsts[p], prs[p], pe_v, lm)
        out_copy(lm, p).start()

    def body(i, carry):
        l = 2 * i
        arrive(l, 0)

        @pl.when(l >= 1)
        def _():
            compute(l - 1, 1)

        arrive(l + 1, 1)
        compute(l, 0)
        return carry

    lax.fori_loop(0, L // 2, body, 0)
    compute(L - 1, 1)
    out_copy(L - 2, 0).wait()
    out_copy(L - 1, 1).wait()


@jax.jit
def kernel(tokens, table, pe):
    mesh = plsc.VectorSubcoreMesh(core_axis_name="c", subcore_axis_name="s")
    params = pltpu.CompilerParams(
        use_tc_tiling_on_sc=True, needs_layout_passes=False)

    k1 = functools.partial(
        pl.kernel, mesh=mesh,
        out_type=jax.ShapeDtypeStruct((VOCAB // 2, 128), jnp.float32),
        scratch_types=[
            pltpu.VMEM((EMBED, 128), jnp.float32),
            pltpu.VMEM((EMBED, 128), jnp.float32),
            pltpu.VMEM((EMBED, 128), jnp.float32),
            pltpu.VMEM((EMBED, 128), jnp.float32),
            pltpu.SemaphoreType.DMA,
            pltpu.SemaphoreType.DMA,
            pltpu.SemaphoreType.DMA,
            pltpu.SemaphoreType.DMA,
        ],
        compiler_params=params,
    )(_k1_body)
    tableT = table.T
    tail128 = lax.slice(tableT, (0, VOCAB - 128), (EMBED, VOCAB))
    scratch = k1(tableT, tail128)

    k2 = functools.partial(
        pl.kernel, mesh=mesh,
        out_type=jax.ShapeDtypeStruct((L, EMBED, B), jnp.float32),
        scratch_types=[
            pltpu.VMEM((128,), jnp.int32),
            pltpu.VMEM((128,), jnp.int32),
            pltpu.VMEM((128,), jnp.int32),
            pltpu.VMEM((128,), jnp.int32),
            pltpu.VMEM((128,), jnp.int32),
            pltpu.VMEM((128,), jnp.int32),
            pltpu.VMEM((128, 128), jnp.float32),
            pltpu.VMEM((128, 128), jnp.float32),
            pltpu.VMEM((EMBED, 128), jnp.float32),
            pltpu.VMEM((EMBED, 128), jnp.float32),
            pltpu.VMEM((L, EMBED), jnp.float32),
            pltpu.SemaphoreType.DMA,
            pltpu.SemaphoreType.DMA,
            pltpu.SemaphoreType.DMA,
            pltpu.SemaphoreType.DMA,
            pltpu.SemaphoreType.DMA,
            pltpu.SemaphoreType.DMA,
        ],
        compiler_params=params,
    )(_k2_body)
    out3 = k2(tokens.T, scratch, pe)
    return jnp.transpose(out3, (2, 0, 1))
